# baseline (speedup 1.0000x reference)
import functools

import jax
import jax.numpy as jnp
from jax import lax
from jax.experimental import pallas as pl
from jax.experimental.pallas import tpu as pltpu
from jax.experimental.pallas import tpu_sc as plsc

N = 10000
E = 320000
NP = 10240
SLEN = 128
EP = 327680
NSTREAM = EP // SLEN
NC, NS = 2, 16
NW = NC * NS
SPW = NSTREAM // NW
CH = 20
ROWS_PER_SUB = NP // NS

F32 = jnp.float32
I32 = jnp.int32


def _deg_body(col_hbm, out_hbm, stage_sp, degv, colbuf, tmp, acc, isem):
    c = lax.axis_index("c")
    s = lax.axis_index("s")
    wid = s * NC + c

    zeros16 = jnp.zeros((16,), F32)
    ones16 = jnp.full((16,), 1.0, F32)

    igets = [
        pltpu.async_copy(col_hbm.at[pl.ds(wid * SPW + k * CH, CH)],
                         colbuf.at[pl.ds(k * CH, CH)], isem)
        for k in range(SPW // CH)
    ]

    def _zero(i, _):
        degv[pl.ds(i * 16, 16)] = zeros16
        return _

    lax.fori_loop(0, NP // 16, _zero, None)

    mask16 = jnp.full((16,), 0xFFFF, I32)

    def _scat(j, _):
        for k in range(4):
            w = colbuf[j, pl.ds(k * 16, 16)]
            plsc.addupdate_scatter(degv, [jnp.bitwise_and(w, mask16)], ones16)
            plsc.addupdate_scatter(degv, [lax.shift_right_logical(w, 16)],
                                   ones16)
        return _

    for k in range(SPW // CH):
        igets[k].wait()
        lax.fori_loop(k * CH, (k + 1) * CH, _scat, None)

    pltpu.sync_copy(degv, stage_sp.at[s])
    plsc.subcore_barrier()

    base = s * ROWS_PER_SUB
    pltpu.sync_copy(stage_sp.at[0, pl.ds(base, ROWS_PER_SUB)], acc)
    rgets = [
        pltpu.async_copy(stage_sp.at[t, pl.ds(base, ROWS_PER_SUB)],
                         tmp.at[t - 1], isem)
        for t in range(1, NS)
    ]
    for t in range(1, NS):
        rgets[t - 1].wait()
        for k in range(ROWS_PER_SUB // 16):
            sl = pl.ds(k * 16, 16)
            acc[sl] = acc[sl] + tmp[t - 1, sl]

    pltpu.sync_copy(acc, out_hbm.at[c, pl.ds(base, ROWS_PER_SUB)])


@functools.cache
def _deg_kernel():
    mesh = plsc.VectorSubcoreMesh(core_axis_name="c", subcore_axis_name="s",
                                  num_cores=NC, num_subcores=NS)
    return pl.kernel(
        _deg_body,
        out_type=jax.ShapeDtypeStruct((NC, NP), F32),
        mesh=mesh,
        compiler_params=pltpu.CompilerParams(needs_layout_passes=False,
                                             use_tc_tiling_on_sc=False),
        scratch_types=[
            pltpu.VMEM_SHARED((NS, NP), F32),
            pltpu.VMEM((NP,), F32),
            pltpu.VMEM((SPW, SLEN // 2), I32),
            pltpu.VMEM((NS - 1, ROWS_PER_SUB), F32),
            pltpu.VMEM((ROWS_PER_SUB,), F32),
            pltpu.SemaphoreType.DMA,
        ],
    )


def _prop_body(row_hbm, col_hbm, h_hbm, degp_hbm,
               out_hbm, dinvp_hbm,
               aggr_sp, hn_sp, colbuf, rowbuf, gbuf, hbuf, dbuf, tbuf,
               packbuf, gsem, ssem, isem):
    c = lax.axis_index("c")
    s = lax.axis_index("s")
    wid = s * NC + c

    igets = []
    for k in range(SPW // CH):
        src = pl.ds(wid * SPW + k * CH, CH)
        dst = pl.ds(k * CH, CH)
        igets.append((
            pltpu.async_copy(col_hbm.at[src], colbuf.at[dst], isem),
            pltpu.async_copy(row_hbm.at[src], rowbuf.at[dst], isem),
        ))

    sl = pl.ds(s * ROWS_PER_SUB, ROWS_PER_SUB)
    hget = pltpu.async_copy(h_hbm.at[sl], hbuf, gsem)
    dget0 = pltpu.async_copy(degp_hbm.at[0, sl], dbuf, gsem)
    dget1 = pltpu.async_copy(degp_hbm.at[1, sl], tbuf, gsem)

    iota16 = lax.iota(I32, 16)
    dbl = jnp.where(iota16 >= 8, 1, 0)
    half16 = jnp.full((16,), 0.5, F32)
    th16 = jnp.full((16,), 1.5, F32)
    magic16 = jnp.full((16,), 0x5f3759df, I32)

    hget.wait()
    dget0.wait()
    dget1.wait()
    for v in range(ROWS_PER_SUB // 16):
        vsl = pl.ds(v * 16, 16)
        d = dbuf[vsl] + tbuf[vsl] + 1.0
        i = plsc.bitcast(d, I32)
        y = plsc.bitcast(magic16 - lax.shift_right_logical(i, 1), F32)
        hd = half16 * d
        for _ in range(3):
            y = y * (th16 - hd * y * y)
        dbuf[vsl] = y

    colidx = jnp.bitwise_and(iota16, 7)
    for v in range(ROWS_PER_SUB // 2):
        ridx = 2 * v + dbl
        dv = plsc.load_gather(dbuf, [ridx])
        hv = plsc.load_gather(hbuf, [ridx, colidx])
        plsc.store_scatter(hbuf, [ridx, colidx], hv * dv)
        packbuf[v // 8, pl.ds((v % 8) * 16, 16)] = dv

    pltpu.sync_copy(hbuf, hn_sp.at[sl])

    @pl.when(c == 0)
    def _():
        pltpu.sync_copy(hbuf, aggr_sp.at[sl])
        pltpu.sync_copy(packbuf, dinvp_hbm.at[pl.ds(s * (ROWS_PER_SUB // 16),
                                                    ROWS_PER_SUB // 16)])

    @pl.when(c == 1)
    def _():
        zeros16 = jnp.zeros((16,), F32)
        for v in range(ROWS_PER_SUB // 2):
            plsc.store_scatter(hbuf, [2 * v + dbl, colidx], zeros16)
        pltpu.sync_copy(hbuf, aggr_sp.at[sl])

    plsc.subcore_barrier()

    gets = []
    puts = []
    for k in range(SPW // CH):
        igets[k][0].wait()
        igets[k][1].wait()
        gets += [
            pltpu.async_copy(hn_sp.at[colbuf.at[j]], gbuf.at[j], gsem)
            for j in range(k * CH, (k + 1) * CH)
        ]
        if k > 0:
            for j in range((k - 1) * CH, k * CH):
                gets[j].wait()
                puts.append(pltpu.async_copy(
                    gbuf.at[j], aggr_sp.at[rowbuf.at[j]], ssem, add=True))
    for j in range(SPW - CH, SPW):
        gets[j].wait()
        puts.append(pltpu.async_copy(
            gbuf.at[j], aggr_sp.at[rowbuf.at[j]], ssem, add=True))
    for d in puts:
        d.wait()

    plsc.subcore_barrier()

    pltpu.sync_copy(aggr_sp.at[sl], out_hbm.at[c, sl])


@functools.cache
def _prop_kernel():
    mesh = plsc.VectorSubcoreMesh(core_axis_name="c", subcore_axis_name="s",
                                  num_cores=NC, num_subcores=NS)
    return pl.kernel(
        _prop_body,
        out_type=(jax.ShapeDtypeStruct((NC, NP, 8), F32),
                  jax.ShapeDtypeStruct((NP // 16, 128), F32)),
        mesh=mesh,
        compiler_params=pltpu.CompilerParams(needs_layout_passes=False,
                                             use_tc_tiling_on_sc=False),
        scratch_types=[
            pltpu.VMEM_SHARED((NP, 8), F32),
            pltpu.VMEM_SHARED((NP, 8), F32),
            pltpu.VMEM((SPW, SLEN), I32),
            pltpu.VMEM((SPW, SLEN), I32),
            pltpu.VMEM((SPW, SLEN, 8), F32),
            pltpu.VMEM((ROWS_PER_SUB, 8), F32),
            pltpu.VMEM((ROWS_PER_SUB,), F32),
            pltpu.VMEM((ROWS_PER_SUB,), F32),
            pltpu.VMEM((ROWS_PER_SUB // 16, 128), F32),
            pltpu.SemaphoreType.DMA,
            pltpu.SemaphoreType.DMA,
            pltpu.SemaphoreType.DMA,
        ],
    )


def _mlp1_body(x_ref, w1, b1, w2, b2, w3, b3, h_ref):
    h = jax.nn.relu(jnp.dot(x_ref[...], w1[...], preferred_element_type=F32)
                    + b1[...][None, :])
    h = jax.nn.relu(jnp.dot(h, w2[...], preferred_element_type=F32)
                    + b2[...][None, :])
    h = jnp.dot(h, w3[...], preferred_element_type=F32) + b3[...][None, :]
    h_ref[0:N, :] = h
    h_ref[N:NP, :] = jnp.zeros((NP - N, 8), F32)


def _fullblk(shape):
    return pl.BlockSpec(shape, lambda *_: (0,) * len(shape))


def _mlp1_call(x, w1, b1, w2, b2, w3, b3):
    return pl.pallas_call(
        _mlp1_body,
        in_specs=[
            _fullblk((N, 128)),
            _fullblk((128, 16)), _fullblk((16,)),
            _fullblk((16, 8)), _fullblk((8,)),
            _fullblk((8, 8)), _fullblk((8,)),
        ],
        out_specs=[_fullblk((NP, 8))],
        out_shape=[jax.ShapeDtypeStruct((NP, 8), F32)],
    )(x, w1, b1, w2, b2, w3, b3)


def _tail_body(x_ref, s_ref, dinvp_ref, batch_ref,
               k1, kb1, k2, kb2, k3, kb3,
               n1, nb1, n2, nb2, n3, nb3,
               g1, gb1, g2, gb2, g3, gb3,
               lo_ref, hi_ref,
               node_ref, dag_ref, z_ref):
    u = s_ref[0:NP // 16, :] + s_ref[NP // 16:NP // 8, :]
    aggrp = dinvp_ref[...] * u

    t = jax.nn.relu(jnp.dot(aggrp, k1[...], preferred_element_type=F32)
                    + kb1[...][None, :])
    t = jax.nn.relu(jnp.dot(t, k2[...], preferred_element_type=F32)
                    + kb2[...][None, :])
    nep = jnp.dot(t, k3[...], preferred_element_type=F32) + kb3[...][None, :]
    ne = jnp.reshape(nep, (NP, 128))[0:N, :]
    node_ref[...] = ne

    m = jax.nn.relu(
        jnp.dot(x_ref[...], n1[0:128, :], preferred_element_type=F32)
        + jnp.dot(ne, n1[128:256, :], preferred_element_type=F32)
        + nb1[...][None, :])
    m = jax.nn.relu(jnp.dot(m, n2[...], preferred_element_type=F32)
                    + nb2[...][None, :])
    m = jnp.dot(m, n3[...], preferred_element_type=F32) + nb3[...][None, :]

    iota_dag = lax.broadcasted_iota(I32, (N, 128), 1)
    onehot = jnp.where(batch_ref[...] == iota_dag, 1.0, 0.0).astype(F32)
    dag = lax.dot_general(
        onehot, m, (((0,), (0,)), ((), ())), preferred_element_type=F32)
    dag_ref[...] = dag

    d = jax.nn.relu(jnp.dot(dag, g1[...], preferred_element_type=F32)
                    + gb1[...][None, :])
    d = jax.nn.relu(jnp.dot(d, g2[...], preferred_element_type=F32)
                    + gb2[...][None, :])
    de = jnp.dot(d, g3[...], preferred_element_type=F32) + gb3[...][None, :]
    iota_env = lax.broadcasted_iota(I32, (16, 128), 1)
    msk = jnp.where((iota_env >= lo_ref[...]) & (iota_env < hi_ref[...]),
                    1.0, 0.0).astype(F32)
    z_ref[...] = jnp.dot(msk, de, preferred_element_type=F32)


def _tail_call(x, s1280, dinvp, batch2d, wts, lo, hi):
    wspecs = [_fullblk(w.shape) for w in wts]
    return pl.pallas_call(
        _tail_body,
        in_specs=[_fullblk((N, 128)), _fullblk((NP // 8, 128)),
                  _fullblk((NP // 16, 128)), _fullblk((N, 1))]
                 + wspecs + [_fullblk((16, 1)), _fullblk((16, 1))],
        out_specs=[
            _fullblk((N, 128)),
            _fullblk((128, 128)),
            _fullblk((16, 128)),
        ],
        out_shape=[
            jax.ShapeDtypeStruct((N, 128), F32),
            jax.ShapeDtypeStruct((128, 128), F32),
            jax.ShapeDtypeStruct((16, 128), F32),
        ],
    )(x, s1280, dinvp, batch2d, *wts, lo, hi)


def kernel(x, edge_index, batch, env_indptr, mlp1, mlp2, mlp_node, mlp_dag):
    pad = jnp.full((2, EP - E), NP - 1, I32)
    ei = jnp.concatenate([edge_index, pad], axis=1)
    row2d = ei[0].reshape(NSTREAM, SLEN)
    col2d = ei[1].reshape(NSTREAM, SLEN)

    batch2d = batch.reshape(N, 1)
    nenv = env_indptr.shape[0] - 1
    lo = jnp.concatenate([env_indptr[:-1],
                          jnp.zeros((16 - nenv,), I32)]).reshape(16, 1)
    hi = jnp.concatenate([env_indptr[1:],
                          jnp.zeros((16 - nenv,), I32)]).reshape(16, 1)

    colpk = col2d[:, 0::2] | (col2d[:, 1::2] << 16)
    degout = _deg_kernel()(colpk)
    w1, b1, w2, b2, w3, b3 = mlp1
    h = _mlp1_call(x, w1, b1, w2, b2, w3, b3)[0]

    s, dinvp = _prop_kernel()(row2d, col2d, h, degout)
    s1280 = s.reshape(NP // 8, 128)

    a1, ab1, a2, ab2, a3, ab3 = mlp2
    eye16 = jnp.eye(16, dtype=F32)
    k1, kb1 = jnp.kron(eye16, a1), jnp.tile(ab1, 16)
    k2, kb2 = jnp.kron(eye16, a2), jnp.tile(ab2, 16)
    k3, kb3 = jnp.kron(eye16, a3), jnp.tile(ab3, 16)
    n1, nb1, n2, nb2, n3, nb3 = mlp_node
    g1, gb1, g2, gb2, g3, gb3 = mlp_dag
    wts = [k1, kb1, k2, kb2, k3, kb3,
           n1, nb1, n2, nb2, n3, nb3,
           g1, gb1, g2, gb2, g3, gb3]
    node_emb, dag128, z16 = _tail_call(
        x, s1280, dinvp, batch2d, wts, lo, hi)

    return node_emb, dag128[:100], z16[:10]


# speedup vs baseline: 1.5666x; 1.5666x over previous
import functools

import jax
import jax.numpy as jnp
from jax import lax
from jax.experimental import pallas as pl
from jax.experimental.pallas import tpu as pltpu
from jax.experimental.pallas import tpu_sc as plsc

N = 10000
E = 320000
NP = 10240
SLEN = 128
EP = 327680
NSTREAM = EP // SLEN
NC, NS = 2, 16
NW = NC * NS
SPW = NSTREAM // NW
CH = 20
ROWS_PER_SUB = NP // NS

F32 = jnp.float32
I32 = jnp.int32


def _deg_body(col_hbm, out_hbm, stage_sp, degv, colbuf, tmp, acc, isem):
    c = lax.axis_index("c")
    s = lax.axis_index("s")
    wid = s * NC + c

    zeros16 = jnp.zeros((16,), F32)
    ones16 = jnp.full((16,), 1.0, F32)

    igets = [
        pltpu.async_copy(col_hbm.at[pl.ds(wid * SPW + k * CH, CH)],
                         colbuf.at[pl.ds(k * CH, CH)], isem)
        for k in range(SPW // CH)
    ]

    def _zero(i, _):
        degv[pl.ds(i * 16, 16)] = zeros16
        return _

    lax.fori_loop(0, NP // 16, _zero, None)

    def _scat(j, _):
        for k in range(8):
            idx = colbuf[j, pl.ds(k * 16, 16)]
            plsc.addupdate_scatter(degv, [idx], ones16)
        return _

    for k in range(SPW // CH):
        igets[k].wait()
        lax.fori_loop(k * CH, (k + 1) * CH, _scat, None)

    pltpu.sync_copy(degv, stage_sp.at[s])
    plsc.subcore_barrier()

    base = s * ROWS_PER_SUB
    pltpu.sync_copy(stage_sp.at[0, pl.ds(base, ROWS_PER_SUB)], acc)
    rgets = [
        pltpu.async_copy(stage_sp.at[t, pl.ds(base, ROWS_PER_SUB)],
                         tmp.at[t - 1], isem)
        for t in range(1, NS)
    ]
    for t in range(1, NS):
        rgets[t - 1].wait()
        for k in range(ROWS_PER_SUB // 16):
            sl = pl.ds(k * 16, 16)
            acc[sl] = acc[sl] + tmp[t - 1, sl]

    pltpu.sync_copy(acc, out_hbm.at[c, pl.ds(base, ROWS_PER_SUB)])


@functools.cache
def _deg_kernel():
    mesh = plsc.VectorSubcoreMesh(core_axis_name="c", subcore_axis_name="s",
                                  num_cores=NC, num_subcores=NS)
    return pl.kernel(
        _deg_body,
        out_type=jax.ShapeDtypeStruct((NC, NP), F32),
        mesh=mesh,
        compiler_params=pltpu.CompilerParams(needs_layout_passes=False,
                                             use_tc_tiling_on_sc=False),
        scratch_types=[
            pltpu.VMEM_SHARED((NS, NP), F32),
            pltpu.VMEM((NP,), F32),
            pltpu.VMEM((SPW, SLEN), I32),
            pltpu.VMEM((NS - 1, ROWS_PER_SUB), F32),
            pltpu.VMEM((ROWS_PER_SUB,), F32),
            pltpu.SemaphoreType.DMA,
        ],
    )


def _prop_body(row_hbm, col_hbm, h_hbm, degp_hbm,
               out_hbm, dinvp_hbm,
               aggr_sp, hn_sp, colbuf, rowbuf, gbuf, hbuf, dbuf, tbuf,
               packbuf, gsem, ssem, isem):
    c = lax.axis_index("c")
    s = lax.axis_index("s")
    wid = s * NC + c

    igets = []
    for k in range(SPW // CH):
        src = pl.ds(wid * SPW + k * CH, CH)
        dst = pl.ds(k * CH, CH)
        igets.append((
            pltpu.async_copy(col_hbm.at[src], colbuf.at[dst], isem),
            pltpu.async_copy(row_hbm.at[src], rowbuf.at[dst], isem),
        ))

    sl = pl.ds(s * ROWS_PER_SUB, ROWS_PER_SUB)
    hget = pltpu.async_copy(h_hbm.at[sl], hbuf, gsem)
    dget0 = pltpu.async_copy(degp_hbm.at[0, sl], dbuf, gsem)
    dget1 = pltpu.async_copy(degp_hbm.at[1, sl], tbuf, gsem)

    iota16 = lax.iota(I32, 16)
    dbl = jnp.where(iota16 >= 8, 1, 0)
    half16 = jnp.full((16,), 0.5, F32)
    th16 = jnp.full((16,), 1.5, F32)
    magic16 = jnp.full((16,), 0x5f3759df, I32)

    hget.wait()
    dget0.wait()
    dget1.wait()
    for v in range(ROWS_PER_SUB // 16):
        vsl = pl.ds(v * 16, 16)
        d = dbuf[vsl] + tbuf[vsl] + 1.0
        i = plsc.bitcast(d, I32)
        y = plsc.bitcast(magic16 - lax.shift_right_logical(i, 1), F32)
        hd = half16 * d
        for _ in range(3):
            y = y * (th16 - hd * y * y)
        dbuf[vsl] = y

    colidx = jnp.bitwise_and(iota16, 7)
    for v in range(ROWS_PER_SUB // 2):
        ridx = 2 * v + dbl
        dv = plsc.load_gather(dbuf, [ridx])
        hv = plsc.load_gather(hbuf, [ridx, colidx])
        plsc.store_scatter(hbuf, [ridx, colidx], hv * dv)
        packbuf[v // 8, pl.ds((v % 8) * 16, 16)] = dv

    pltpu.sync_copy(hbuf, hn_sp.at[sl])

    @pl.when(c == 0)
    def _():
        pltpu.sync_copy(hbuf, aggr_sp.at[sl])
        pltpu.sync_copy(packbuf, dinvp_hbm.at[pl.ds(s * (ROWS_PER_SUB // 16),
                                                    ROWS_PER_SUB // 16)])

    @pl.when(c == 1)
    def _():
        zeros16 = jnp.zeros((16,), F32)
        for v in range(ROWS_PER_SUB // 2):
            plsc.store_scatter(hbuf, [2 * v + dbl, colidx], zeros16)
        pltpu.sync_copy(hbuf, aggr_sp.at[sl])

    plsc.subcore_barrier()

    gets = []
    puts = []
    for k in range(SPW // CH):
        igets[k][0].wait()
        igets[k][1].wait()
        gets += [
            pltpu.async_copy(hn_sp.at[colbuf.at[j]], gbuf.at[j], gsem)
            for j in range(k * CH, (k + 1) * CH)
        ]
        if k > 0:
            for j in range((k - 1) * CH, k * CH):
                gets[j].wait()
                puts.append(pltpu.async_copy(
                    gbuf.at[j], aggr_sp.at[rowbuf.at[j]], ssem, add=True))
    for j in range(SPW - CH, SPW):
        gets[j].wait()
        puts.append(pltpu.async_copy(
            gbuf.at[j], aggr_sp.at[rowbuf.at[j]], ssem, add=True))
    for d in puts:
        d.wait()

    plsc.subcore_barrier()

    pltpu.sync_copy(aggr_sp.at[sl], out_hbm.at[c, sl])


@functools.cache
def _prop_kernel():
    mesh = plsc.VectorSubcoreMesh(core_axis_name="c", subcore_axis_name="s",
                                  num_cores=NC, num_subcores=NS)
    return pl.kernel(
        _prop_body,
        out_type=(jax.ShapeDtypeStruct((NC, NP, 8), F32),
                  jax.ShapeDtypeStruct((NP // 16, 128), F32)),
        mesh=mesh,
        compiler_params=pltpu.CompilerParams(needs_layout_passes=False,
                                             use_tc_tiling_on_sc=False),
        scratch_types=[
            pltpu.VMEM_SHARED((NP, 8), F32),
            pltpu.VMEM_SHARED((NP, 8), F32),
            pltpu.VMEM((SPW, SLEN), I32),
            pltpu.VMEM((SPW, SLEN), I32),
            pltpu.VMEM((SPW, SLEN, 8), F32),
            pltpu.VMEM((ROWS_PER_SUB, 8), F32),
            pltpu.VMEM((ROWS_PER_SUB,), F32),
            pltpu.VMEM((ROWS_PER_SUB,), F32),
            pltpu.VMEM((ROWS_PER_SUB // 16, 128), F32),
            pltpu.SemaphoreType.DMA,
            pltpu.SemaphoreType.DMA,
            pltpu.SemaphoreType.DMA,
        ],
    )


def _mlp1_body(x_ref, w1, b1, w2, b2, w3, b3, h_ref):
    h = jax.nn.relu(jnp.dot(x_ref[...], w1[...], preferred_element_type=F32)
                    + b1[...][None, :])
    h = jax.nn.relu(jnp.dot(h, w2[...], preferred_element_type=F32)
                    + b2[...][None, :])
    h = jnp.dot(h, w3[...], preferred_element_type=F32) + b3[...][None, :]
    h_ref[0:N, :] = h
    h_ref[N:NP, :] = jnp.zeros((NP - N, 8), F32)


def _fullblk(shape):
    return pl.BlockSpec(shape, lambda *_: (0,) * len(shape))


def _mlp1_call(x, w1, b1, w2, b2, w3, b3):
    return pl.pallas_call(
        _mlp1_body,
        in_specs=[
            _fullblk((N, 128)),
            _fullblk((128, 16)), _fullblk((16,)),
            _fullblk((16, 8)), _fullblk((8,)),
            _fullblk((8, 8)), _fullblk((8,)),
        ],
        out_specs=[_fullblk((NP, 8))],
        out_shape=[jax.ShapeDtypeStruct((NP, 8), F32)],
    )(x, w1, b1, w2, b2, w3, b3)


def _tail_body(x_ref, s_ref, dinvp_ref, batch_ref,
               k1, kb1, k2, kb2, k3, kb3,
               n1, nb1, n2, nb2, n3, nb3,
               g1, gb1, g2, gb2, g3, gb3,
               lo_ref, hi_ref,
               node_ref, dag_ref, z_ref):
    u = s_ref[0:NP // 16, :] + s_ref[NP // 16:NP // 8, :]
    aggrp = dinvp_ref[...] * u

    t = jax.nn.relu(jnp.dot(aggrp, k1[...], preferred_element_type=F32)
                    + kb1[...][None, :])
    t = jax.nn.relu(jnp.dot(t, k2[...], preferred_element_type=F32)
                    + kb2[...][None, :])
    nep = jnp.dot(t, k3[...], preferred_element_type=F32) + kb3[...][None, :]
    ne = jnp.reshape(nep, (NP, 128))[0:N, :]
    node_ref[...] = ne

    m = jax.nn.relu(
        jnp.dot(x_ref[...], n1[0:128, :], preferred_element_type=F32)
        + jnp.dot(ne, n1[128:256, :], preferred_element_type=F32)
        + nb1[...][None, :])
    m = jax.nn.relu(jnp.dot(m, n2[...], preferred_element_type=F32)
                    + nb2[...][None, :])
    m = jnp.dot(m, n3[...], preferred_element_type=F32) + nb3[...][None, :]

    iota_dag = lax.broadcasted_iota(I32, (N, 128), 1)
    onehot = jnp.where(batch_ref[...] == iota_dag, 1.0, 0.0).astype(F32)
    dag = lax.dot_general(
        onehot, m, (((0,), (0,)), ((), ())), preferred_element_type=F32)
    dag_ref[...] = dag

    d = jax.nn.relu(jnp.dot(dag, g1[...], preferred_element_type=F32)
                    + gb1[...][None, :])
    d = jax.nn.relu(jnp.dot(d, g2[...], preferred_element_type=F32)
                    + gb2[...][None, :])
    de = jnp.dot(d, g3[...], preferred_element_type=F32) + gb3[...][None, :]
    iota_env = lax.broadcasted_iota(I32, (16, 128), 1)
    msk = jnp.where((iota_env >= lo_ref[...]) & (iota_env < hi_ref[...]),
                    1.0, 0.0).astype(F32)
    z_ref[...] = jnp.dot(msk, de, preferred_element_type=F32)


def _tail_call(x, s1280, dinvp, batch2d, wts, lo, hi):
    wspecs = [_fullblk(w.shape) for w in wts]
    return pl.pallas_call(
        _tail_body,
        in_specs=[_fullblk((N, 128)), _fullblk((NP // 8, 128)),
                  _fullblk((NP // 16, 128)), _fullblk((N, 1))]
                 + wspecs + [_fullblk((16, 1)), _fullblk((16, 1))],
        out_specs=[
            _fullblk((N, 128)),
            _fullblk((128, 128)),
            _fullblk((16, 128)),
        ],
        out_shape=[
            jax.ShapeDtypeStruct((N, 128), F32),
            jax.ShapeDtypeStruct((128, 128), F32),
            jax.ShapeDtypeStruct((16, 128), F32),
        ],
    )(x, s1280, dinvp, batch2d, *wts, lo, hi)


def kernel(x, edge_index, batch, env_indptr, mlp1, mlp2, mlp_node, mlp_dag):
    pad = jnp.full((2, EP - E), NP - 1, I32)
    ei = jnp.concatenate([edge_index, pad], axis=1)
    row2d = ei[0].reshape(NSTREAM, SLEN)
    col2d = ei[1].reshape(NSTREAM, SLEN)

    batch2d = batch.reshape(N, 1)
    nenv = env_indptr.shape[0] - 1
    lo = jnp.concatenate([env_indptr[:-1],
                          jnp.zeros((16 - nenv,), I32)]).reshape(16, 1)
    hi = jnp.concatenate([env_indptr[1:],
                          jnp.zeros((16 - nenv,), I32)]).reshape(16, 1)

    degout = _deg_kernel()(col2d)
    w1, b1, w2, b2, w3, b3 = mlp1
    h = _mlp1_call(x, w1, b1, w2, b2, w3, b3)[0]

    s, dinvp = _prop_kernel()(row2d, col2d, h, degout)
    s1280 = s.reshape(NP // 8, 128)

    a1, ab1, a2, ab2, a3, ab3 = mlp2
    eye16 = jnp.eye(16, dtype=F32)
    k1, kb1 = jnp.kron(eye16, a1), jnp.tile(ab1, 16)
    k2, kb2 = jnp.kron(eye16, a2), jnp.tile(ab2, 16)
    k3, kb3 = jnp.kron(eye16, a3), jnp.tile(ab3, 16)
    n1, nb1, n2, nb2, n3, nb3 = mlp_node
    g1, gb1, g2, gb2, g3, gb3 = mlp_dag
    wts = [k1, kb1, k2, kb2, k3, kb3,
           n1, nb1, n2, nb2, n3, nb3,
           g1, gb1, g2, gb2, g3, gb3]
    node_emb, dag128, z16 = _tail_call(
        x, s1280, dinvp, batch2d, wts, lo, hi)

    return node_emb, dag128[:100], z16[:10]


# speedup vs baseline: 1.7091x; 1.0910x over previous
import functools

import jax
import jax.numpy as jnp
from jax import lax
from jax.experimental import pallas as pl
from jax.experimental.pallas import tpu as pltpu
from jax.experimental.pallas import tpu_sc as plsc

N = 10000
E = 320000
NP = 10240
SLEN = 128
EP = 327680
NSTREAM = EP // SLEN
NC, NS = 2, 16
NW = NC * NS
SPW = NSTREAM // NW
CH = 20
ROWS_PER_SUB = NP // NS

F32 = jnp.float32
I32 = jnp.int32


def _deg_body(ei_hbm, out_hbm, stage_sp, degv, colbuf, tmp, acc, isem):
    c = lax.axis_index("c")
    s = lax.axis_index("s")
    wid = s * NC + c

    zeros16 = jnp.zeros((16,), F32)
    ones16 = jnp.full((16,), 1.0, F32)

    ECH = E // NW // 5
    igets = [
        pltpu.async_copy(ei_hbm.at[1, pl.ds(wid * (E // NW) + k * ECH, ECH)],
                         colbuf.at[pl.ds(k * ECH, ECH)], isem)
        for k in range(5)
    ]

    def _zero(i, _):
        degv[pl.ds(i * 16, 16)] = zeros16
        return _

    lax.fori_loop(0, NP // 16, _zero, None)

    def _scat(v, _):
        idx = colbuf[pl.ds(v * 16, 16)]
        plsc.addupdate_scatter(degv, [idx], ones16)
        return _

    for k in range(5):
        igets[k].wait()
        lax.fori_loop(k * (ECH // 16), (k + 1) * (ECH // 16), _scat, None)

    pltpu.sync_copy(degv, stage_sp.at[s])
    plsc.subcore_barrier()

    base = s * ROWS_PER_SUB
    pltpu.sync_copy(stage_sp.at[0, pl.ds(base, ROWS_PER_SUB)], acc)
    rgets = [
        pltpu.async_copy(stage_sp.at[t, pl.ds(base, ROWS_PER_SUB)],
                         tmp.at[t - 1], isem)
        for t in range(1, NS)
    ]
    for t in range(1, NS):
        rgets[t - 1].wait()
        for k in range(ROWS_PER_SUB // 16):
            sl = pl.ds(k * 16, 16)
            acc[sl] = acc[sl] + tmp[t - 1, sl]

    pltpu.sync_copy(acc, out_hbm.at[c, pl.ds(base, ROWS_PER_SUB)])


@functools.cache
def _deg_kernel():
    mesh = plsc.VectorSubcoreMesh(core_axis_name="c", subcore_axis_name="s",
                                  num_cores=NC, num_subcores=NS)
    return pl.kernel(
        _deg_body,
        out_type=jax.ShapeDtypeStruct((NC, NP), F32),
        mesh=mesh,
        compiler_params=pltpu.CompilerParams(needs_layout_passes=False,
                                             use_tc_tiling_on_sc=False),
        scratch_types=[
            pltpu.VMEM_SHARED((NS, NP), F32),
            pltpu.VMEM((NP,), F32),
            pltpu.VMEM((E // NW,), I32),
            pltpu.VMEM((NS - 1, ROWS_PER_SUB), F32),
            pltpu.VMEM((ROWS_PER_SUB,), F32),
            pltpu.SemaphoreType.DMA,
        ],
    )


def _prop_body(row_hbm, col_hbm, h_hbm, degp_hbm,
               out_hbm, dinvp_hbm,
               aggr_sp, hn_sp, colbuf, rowbuf, gbuf, hbuf, dbuf, tbuf,
               packbuf, gsem, ssem, isem):
    c = lax.axis_index("c")
    s = lax.axis_index("s")
    wid = s * NC + c

    igets = []
    for k in range(SPW // CH):
        src = pl.ds(wid * SPW + k * CH, CH)
        dst = pl.ds(k * CH, CH)
        igets.append((
            pltpu.async_copy(col_hbm.at[src], colbuf.at[dst], isem),
            pltpu.async_copy(row_hbm.at[src], rowbuf.at[dst], isem),
        ))

    sl = pl.ds(s * ROWS_PER_SUB, ROWS_PER_SUB)
    hget = pltpu.async_copy(h_hbm.at[sl], hbuf, gsem)
    dget0 = pltpu.async_copy(degp_hbm.at[0, sl], dbuf, gsem)
    dget1 = pltpu.async_copy(degp_hbm.at[1, sl], tbuf, gsem)

    iota16 = lax.iota(I32, 16)
    dbl = jnp.where(iota16 >= 8, 1, 0)
    half16 = jnp.full((16,), 0.5, F32)
    th16 = jnp.full((16,), 1.5, F32)
    magic16 = jnp.full((16,), 0x5f3759df, I32)

    hget.wait()
    dget0.wait()
    dget1.wait()
    for v in range(ROWS_PER_SUB // 16):
        vsl = pl.ds(v * 16, 16)
        d = dbuf[vsl] + tbuf[vsl] + 1.0
        i = plsc.bitcast(d, I32)
        y = plsc.bitcast(magic16 - lax.shift_right_logical(i, 1), F32)
        hd = half16 * d
        for _ in range(3):
            y = y * (th16 - hd * y * y)
        dbuf[vsl] = y

    colidx = jnp.bitwise_and(iota16, 7)
    for v in range(ROWS_PER_SUB // 2):
        ridx = 2 * v + dbl
        dv = plsc.load_gather(dbuf, [ridx])
        hv = plsc.load_gather(hbuf, [ridx, colidx])
        plsc.store_scatter(hbuf, [ridx, colidx], hv * dv)
        packbuf[v // 8, pl.ds((v % 8) * 16, 16)] = dv

    pltpu.sync_copy(hbuf, hn_sp.at[sl])

    @pl.when(c == 0)
    def _():
        pltpu.sync_copy(hbuf, aggr_sp.at[sl])
        pltpu.sync_copy(packbuf, dinvp_hbm.at[pl.ds(s * (ROWS_PER_SUB // 16),
                                                    ROWS_PER_SUB // 16)])

    @pl.when(c == 1)
    def _():
        zeros16 = jnp.zeros((16,), F32)
        for v in range(ROWS_PER_SUB // 2):
            plsc.store_scatter(hbuf, [2 * v + dbl, colidx], zeros16)
        pltpu.sync_copy(hbuf, aggr_sp.at[sl])

    plsc.subcore_barrier()

    gets = []
    puts = []
    for k in range(SPW // CH):
        igets[k][0].wait()
        igets[k][1].wait()
        gets += [
            pltpu.async_copy(hn_sp.at[colbuf.at[j]], gbuf.at[j], gsem)
            for j in range(k * CH, (k + 1) * CH)
        ]
        if k > 0:
            for j in range((k - 1) * CH, k * CH):
                gets[j].wait()
                puts.append(pltpu.async_copy(
                    gbuf.at[j], aggr_sp.at[rowbuf.at[j]], ssem, add=True))
    for j in range(SPW - CH, SPW):
        gets[j].wait()
        puts.append(pltpu.async_copy(
            gbuf.at[j], aggr_sp.at[rowbuf.at[j]], ssem, add=True))
    for d in puts:
        d.wait()

    plsc.subcore_barrier()

    pltpu.sync_copy(aggr_sp.at[sl], out_hbm.at[c, sl])


@functools.cache
def _prop_kernel():
    mesh = plsc.VectorSubcoreMesh(core_axis_name="c", subcore_axis_name="s",
                                  num_cores=NC, num_subcores=NS)
    return pl.kernel(
        _prop_body,
        out_type=(jax.ShapeDtypeStruct((NC, NP, 8), F32),
                  jax.ShapeDtypeStruct((NP // 16, 128), F32)),
        mesh=mesh,
        compiler_params=pltpu.CompilerParams(needs_layout_passes=False,
                                             use_tc_tiling_on_sc=False),
        scratch_types=[
            pltpu.VMEM_SHARED((NP, 8), F32),
            pltpu.VMEM_SHARED((NP, 8), F32),
            pltpu.VMEM((SPW, SLEN), I32),
            pltpu.VMEM((SPW, SLEN), I32),
            pltpu.VMEM((SPW, SLEN, 8), F32),
            pltpu.VMEM((ROWS_PER_SUB, 8), F32),
            pltpu.VMEM((ROWS_PER_SUB,), F32),
            pltpu.VMEM((ROWS_PER_SUB,), F32),
            pltpu.VMEM((ROWS_PER_SUB // 16, 128), F32),
            pltpu.SemaphoreType.DMA,
            pltpu.SemaphoreType.DMA,
            pltpu.SemaphoreType.DMA,
        ],
    )


def _mlp1_body(x_ref, w1, b1, w2, b2, w3, b3, h_ref):
    h = jax.nn.relu(jnp.dot(x_ref[...], w1[...], preferred_element_type=F32)
                    + b1[...][None, :])
    h = jax.nn.relu(jnp.dot(h, w2[...], preferred_element_type=F32)
                    + b2[...][None, :])
    h = jnp.dot(h, w3[...], preferred_element_type=F32) + b3[...][None, :]
    h_ref[0:N, :] = h
    h_ref[N:NP, :] = jnp.zeros((NP - N, 8), F32)


def _fullblk(shape):
    return pl.BlockSpec(shape, lambda *_: (0,) * len(shape))


def _mlp1_call(x, w1, b1, w2, b2, w3, b3):
    return pl.pallas_call(
        _mlp1_body,
        in_specs=[
            _fullblk((N, 128)),
            _fullblk((128, 16)), _fullblk((16,)),
            _fullblk((16, 8)), _fullblk((8,)),
            _fullblk((8, 8)), _fullblk((8,)),
        ],
        out_specs=[_fullblk((NP, 8))],
        out_shape=[jax.ShapeDtypeStruct((NP, 8), F32)],
    )(x, w1, b1, w2, b2, w3, b3)


def _tail_body(x_ref, s_ref, dinvp_ref, batch_ref,
               k1, kb1, k2, kb2, k3, kb3,
               n1, nb1, n2, nb2, n3, nb3,
               g1, gb1, g2, gb2, g3, gb3,
               lo_ref, hi_ref,
               node_ref, dag_ref, z_ref):
    u = s_ref[0:NP // 16, :] + s_ref[NP // 16:NP // 8, :]
    aggrp = dinvp_ref[...] * u

    t = jax.nn.relu(jnp.dot(aggrp, k1[...], preferred_element_type=F32)
                    + kb1[...][None, :])
    t = jax.nn.relu(jnp.dot(t, k2[...], preferred_element_type=F32)
                    + kb2[...][None, :])
    nep = jnp.dot(t, k3[...], preferred_element_type=F32) + kb3[...][None, :]
    ne = jnp.reshape(nep, (NP, 128))[0:N, :]
    node_ref[...] = ne

    m = jax.nn.relu(
        jnp.dot(x_ref[...], n1[0:128, :], preferred_element_type=F32)
        + jnp.dot(ne, n1[128:256, :], preferred_element_type=F32)
        + nb1[...][None, :])
    m = jax.nn.relu(jnp.dot(m, n2[...], preferred_element_type=F32)
                    + nb2[...][None, :])
    m = jnp.dot(m, n3[...], preferred_element_type=F32) + nb3[...][None, :]

    iota_dag = lax.broadcasted_iota(I32, (N, 128), 1)
    onehot = jnp.where(batch_ref[...] == iota_dag, 1.0, 0.0).astype(F32)
    dag = lax.dot_general(
        onehot, m, (((0,), (0,)), ((), ())), preferred_element_type=F32)
    dag_ref[...] = dag

    d = jax.nn.relu(jnp.dot(dag, g1[...], preferred_element_type=F32)
                    + gb1[...][None, :])
    d = jax.nn.relu(jnp.dot(d, g2[...], preferred_element_type=F32)
                    + gb2[...][None, :])
    de = jnp.dot(d, g3[...], preferred_element_type=F32) + gb3[...][None, :]
    iota_env = lax.broadcasted_iota(I32, (16, 128), 1)
    msk = jnp.where((iota_env >= lo_ref[...]) & (iota_env < hi_ref[...]),
                    1.0, 0.0).astype(F32)
    z_ref[...] = jnp.dot(msk, de, preferred_element_type=F32)


def _tail_call(x, s1280, dinvp, batch2d, wts, lo, hi):
    wspecs = [_fullblk(w.shape) for w in wts]
    return pl.pallas_call(
        _tail_body,
        in_specs=[_fullblk((N, 128)), _fullblk((NP // 8, 128)),
                  _fullblk((NP // 16, 128)), _fullblk((N, 1))]
                 + wspecs + [_fullblk((16, 1)), _fullblk((16, 1))],
        out_specs=[
            _fullblk((N, 128)),
            _fullblk((128, 128)),
            _fullblk((16, 128)),
        ],
        out_shape=[
            jax.ShapeDtypeStruct((N, 128), F32),
            jax.ShapeDtypeStruct((128, 128), F32),
            jax.ShapeDtypeStruct((16, 128), F32),
        ],
    )(x, s1280, dinvp, batch2d, *wts, lo, hi)


def kernel(x, edge_index, batch, env_indptr, mlp1, mlp2, mlp_node, mlp_dag):
    pad = jnp.full((2, EP - E), NP - 1, I32)
    ei = jnp.concatenate([edge_index, pad], axis=1)
    row2d = ei[0].reshape(NSTREAM, SLEN)
    col2d = ei[1].reshape(NSTREAM, SLEN)

    batch2d = batch.reshape(N, 1)
    nenv = env_indptr.shape[0] - 1
    lo = jnp.concatenate([env_indptr[:-1],
                          jnp.zeros((16 - nenv,), I32)]).reshape(16, 1)
    hi = jnp.concatenate([env_indptr[1:],
                          jnp.zeros((16 - nenv,), I32)]).reshape(16, 1)

    degout = _deg_kernel()(edge_index)
    w1, b1, w2, b2, w3, b3 = mlp1
    h = _mlp1_call(x, w1, b1, w2, b2, w3, b3)[0]

    s, dinvp = _prop_kernel()(row2d, col2d, h, degout)
    s1280 = s.reshape(NP // 8, 128)

    a1, ab1, a2, ab2, a3, ab3 = mlp2
    eye16 = jnp.eye(16, dtype=F32)
    k1, kb1 = jnp.kron(eye16, a1), jnp.tile(ab1, 16)
    k2, kb2 = jnp.kron(eye16, a2), jnp.tile(ab2, 16)
    k3, kb3 = jnp.kron(eye16, a3), jnp.tile(ab3, 16)
    n1, nb1, n2, nb2, n3, nb3 = mlp_node
    g1, gb1, g2, gb2, g3, gb3 = mlp_dag
    wts = [k1, kb1, k2, kb2, k3, kb3,
           n1, nb1, n2, nb2, n3, nb3,
           g1, gb1, g2, gb2, g3, gb3]
    node_emb, dag128, z16 = _tail_call(
        x, s1280, dinvp, batch2d, wts, lo, hi)

    return node_emb, dag128[:100], z16[:10]
